# Initial kernel scaffold; baseline (speedup 1.0000x reference)
import jax
import jax.numpy as jnp
from jax.experimental import pallas as pl


def kernel(input_ids, emb_table, glove_table):
    raise NotImplementedError("write your pallas kernel here")


# speedup vs baseline: 2.7882x; 2.7882x over previous
import functools

import jax
import jax.numpy as jnp
from jax import lax
from jax.experimental import pallas as pl
from jax.experimental.pallas import tpu as pltpu
from jax.experimental.pallas import tpu_sc as plsc

_NC = 2
_NS = 16
_NW = _NC * _NS
_LANES = 16

_B = 4096
_L = 50
_D = 128
_N = _B * _L
_PER_W = _N // _NW
_K = 128
_CHUNKS = _PER_W // _K


def _tanh_vec(x):
    ax = jnp.abs(x)
    e = jnp.exp(ax + ax)
    t = 1.0 - 2.0 / (e + 1.0)
    return jnp.sign(x) * t


def _body(ids_hbm, emb_hbm, glove_hbm, out_hbm, idx_v, emb_v, glove_v,
          sem_e, sem_g):
    wid = lax.axis_index("s") * _NC + lax.axis_index("c")

    def chunk_body(c, carry):
        base = wid * _PER_W + c * _K
        pltpu.sync_copy(ids_hbm.at[pl.ds(base, _K)], idx_v)
        cp_e = pltpu.async_copy(emb_hbm.at[idx_v], emb_v, sem_e)
        cp_g = pltpu.async_copy(glove_hbm.at[idx_v], glove_v, sem_g)
        cp_e.wait()

        def row_body(r, carry2):
            for j in range(_D // _LANES):
                sl = pl.ds(j * _LANES, _LANES)
                emb_v[r, sl] = _tanh_vec(emb_v[r, sl])
            return carry2

        lax.fori_loop(0, _K, row_body, 0)
        cp_g.wait()
        pltpu.sync_copy(emb_v, out_hbm.at[pl.ds(base, _K), pl.ds(0, _D)])
        pltpu.sync_copy(glove_v, out_hbm.at[pl.ds(base, _K), pl.ds(_D, _D)])
        return carry

    lax.fori_loop(0, _CHUNKS, chunk_body, 0)


@functools.partial(jax.jit, static_argnums=())
def _run(ids_flat, emb_table, glove_table):
    mesh = plsc.VectorSubcoreMesh(
        core_axis_name="c", subcore_axis_name="s",
        num_cores=_NC, num_subcores=_NS)
    f = pl.kernel(
        _body,
        out_type=jax.ShapeDtypeStruct((_N, 2 * _D), jnp.float32),
        mesh=mesh,
        scratch_types=[
            pltpu.VMEM((_K,), jnp.int32),
            pltpu.VMEM((_K, _D), jnp.float32),
            pltpu.VMEM((_K, _D), jnp.float32),
            pltpu.SemaphoreType.DMA,
            pltpu.SemaphoreType.DMA,
        ],
    )
    return f(ids_flat, emb_table, glove_table)


def kernel(input_ids, emb_table, glove_table):
    ids_flat = input_ids.reshape(-1).astype(jnp.int32)
    out = _run(ids_flat, emb_table, glove_table)
    return out.reshape(_B, _L, 2 * _D)


# speedup vs baseline: 3.3621x; 1.2059x over previous
import functools

import jax
import jax.numpy as jnp
from jax import lax
from jax.experimental import pallas as pl
from jax.experimental.pallas import tpu as pltpu
from jax.experimental.pallas import tpu_sc as plsc

_NC = 2
_NS = 16
_NW = _NC * _NS
_LANES = 16

_B = 4096
_L = 50
_D = 128
_N = _B * _L
_PER_W = _N // _NW
_K = 128
_CHUNKS = _PER_W // _K
_PAIRS = _CHUNKS // 2


def _tanh_vec(x):
    e = jnp.exp(x + x)
    return 1.0 - 2.0 / (e + 1.0)


def _body(ids_hbm, emb_hbm, glove_hbm, out_hbm, idx2, emb2, glove2,
          gsem0, gsem1, ssem0, ssem1):
    wid = lax.axis_index("s") * _NC + lax.axis_index("c")
    w0 = wid * _PER_W

    def start_gathers(c, b, gsem):
        pltpu.sync_copy(ids_hbm.at[pl.ds(w0 + c * _K, _K)], idx2.at[b])
        pltpu.async_copy(emb_hbm.at[idx2.at[b]], emb2.at[b], gsem)
        pltpu.async_copy(glove_hbm.at[idx2.at[b]], glove2.at[b], gsem)

    def wait_gathers(b, gsem):
        pltpu.make_async_copy(emb_hbm.at[idx2.at[b]], emb2.at[b], gsem).wait()
        pltpu.make_async_copy(glove_hbm.at[idx2.at[b]], glove2.at[b],
                              gsem).wait()

    def start_stores(c, b, ssem):
        base = w0 + c * _K
        pltpu.async_copy(emb2.at[b], out_hbm.at[pl.ds(base, _K), pl.ds(0, _D)],
                         ssem)
        pltpu.async_copy(glove2.at[b],
                         out_hbm.at[pl.ds(base, _K), pl.ds(_D, _D)], ssem)

    def wait_stores(b, ssem):
        pltpu.make_async_copy(emb2.at[b],
                              out_hbm.at[pl.ds(0, _K), pl.ds(0, _D)],
                              ssem).wait()
        pltpu.make_async_copy(glove2.at[b],
                              out_hbm.at[pl.ds(0, _K), pl.ds(_D, _D)],
                              ssem).wait()

    def tanh_chunk(b):
        def row_body(r, carry):
            for j in range(_D // _LANES):
                sl = pl.ds(j * _LANES, _LANES)
                emb2[b, r, sl] = _tanh_vec(emb2[b, r, sl])
            return carry

        lax.fori_loop(0, _K, row_body, 0)

    start_gathers(0, 0, gsem0)

    def pair_body(i, carry):
        c0 = i * 2
        wait_gathers(0, gsem0)

        @pl.when(i > 0)
        def _():
            wait_stores(1, ssem1)

        start_gathers(c0 + 1, 1, gsem1)
        tanh_chunk(0)
        start_stores(c0, 0, ssem0)

        wait_gathers(1, gsem1)

        @pl.when(i < _PAIRS - 1)
        def _():
            wait_stores(0, ssem0)
            start_gathers(c0 + 2, 0, gsem0)

        tanh_chunk(1)
        start_stores(c0 + 1, 1, ssem1)
        return carry

    lax.fori_loop(0, _PAIRS, pair_body, 0)
    wait_stores(0, ssem0)
    wait_stores(1, ssem1)


@jax.jit
def _run(ids_flat, emb_table, glove_table):
    mesh = plsc.VectorSubcoreMesh(
        core_axis_name="c", subcore_axis_name="s",
        num_cores=_NC, num_subcores=_NS)
    f = pl.kernel(
        _body,
        out_type=jax.ShapeDtypeStruct((_N, 2 * _D), jnp.float32),
        mesh=mesh,
        scratch_types=[
            pltpu.VMEM((2, _K), jnp.int32),
            pltpu.VMEM((2, _K, _D), jnp.float32),
            pltpu.VMEM((2, _K, _D), jnp.float32),
            pltpu.SemaphoreType.DMA,
            pltpu.SemaphoreType.DMA,
            pltpu.SemaphoreType.DMA,
            pltpu.SemaphoreType.DMA,
        ],
    )
    return f(ids_flat, emb_table, glove_table)


def kernel(input_ids, emb_table, glove_table):
    ids_flat = input_ids.reshape(-1).astype(jnp.int32)
    out = _run(ids_flat, emb_table, glove_table)
    return out.reshape(_B, _L, 2 * _D)


# speedup vs baseline: 5.7327x; 1.7051x over previous
import jax
import jax.numpy as jnp
from jax import lax
from jax.experimental import pallas as pl
from jax.experimental.pallas import tpu as pltpu
from jax.experimental.pallas import tpu_sc as plsc

_NC = 2
_NS = 16
_NW = _NC * _NS
_LANES = 16

_B = 4096
_L = 50
_D = 128
_RPW = _B // _NW
_R = 4
_CH = _RPW // _R
_PAIRS = _CH // 2


def _tanh_vec(x):
    e = jnp.exp(x + x)
    return 1.0 - 2.0 / (e + 1.0)


def _body(ids_hbm, emb_hbm, glove_hbm, out_hbm, idx_all, emb2, glove2,
          gsem0, gsem1, ssem0, ssem1):
    wid = lax.axis_index("s") * _NC + lax.axis_index("c")
    r0w = wid * _RPW

    pltpu.sync_copy(ids_hbm.at[pl.ds(r0w, _RPW), :], idx_all)

    def idx_ref(c, r):
        return idx_all.at[c * _R + r]

    def start_gathers(c, b, gsem):
        for r in range(_R):
            pltpu.async_copy(emb_hbm.at[idx_ref(c, r)], emb2.at[b, r], gsem)
            pltpu.async_copy(glove_hbm.at[idx_ref(c, r)], glove2.at[b, r],
                             gsem)

    def wait_gathers(b, gsem):
        for r in range(_R):
            pltpu.make_async_copy(emb_hbm.at[idx_ref(0, r)], emb2.at[b, r],
                                  gsem).wait()
            pltpu.make_async_copy(glove_hbm.at[idx_ref(0, r)],
                                  glove2.at[b, r], gsem).wait()

    def start_stores(c, b, ssem):
        base = r0w + c * _R
        pltpu.async_copy(
            emb2.at[b], out_hbm.at[pl.ds(base, _R), :, pl.ds(0, _D)], ssem)
        pltpu.async_copy(
            glove2.at[b], out_hbm.at[pl.ds(base, _R), :, pl.ds(_D, _D)], ssem)

    def wait_stores(b, ssem):
        pltpu.make_async_copy(
            emb2.at[b], out_hbm.at[pl.ds(0, _R), :, pl.ds(0, _D)],
            ssem).wait()
        pltpu.make_async_copy(
            glove2.at[b], out_hbm.at[pl.ds(0, _R), :, pl.ds(_D, _D)],
            ssem).wait()

    def tanh_chunk(b):
        for r in range(_R):
            def l_body(l, carry):
                for j in range(_D // _LANES):
                    sl = pl.ds(j * _LANES, _LANES)
                    emb2[b, r, l, sl] = _tanh_vec(emb2[b, r, l, sl])
                return carry

            lax.fori_loop(0, _L, l_body, 0)

    start_gathers(0, 0, gsem0)

    def pair_body(i, carry):
        c0 = i * 2
        wait_gathers(0, gsem0)

        @pl.when(i > 0)
        def _():
            wait_stores(1, ssem1)

        start_gathers(c0 + 1, 1, gsem1)
        tanh_chunk(0)
        start_stores(c0, 0, ssem0)

        wait_gathers(1, gsem1)

        @pl.when(i < _PAIRS - 1)
        def _():
            wait_stores(0, ssem0)
            start_gathers(c0 + 2, 0, gsem0)

        tanh_chunk(1)
        start_stores(c0 + 1, 1, ssem1)
        return carry

    lax.fori_loop(0, _PAIRS, pair_body, 0)
    wait_stores(0, ssem0)
    wait_stores(1, ssem1)


@jax.jit
def _run(ids, emb_table, glove_table):
    mesh = plsc.VectorSubcoreMesh(
        core_axis_name="c", subcore_axis_name="s",
        num_cores=_NC, num_subcores=_NS)
    f = pl.kernel(
        _body,
        out_type=jax.ShapeDtypeStruct((_B, _L, 2 * _D), jnp.float32),
        mesh=mesh,
        scratch_types=[
            pltpu.VMEM((_RPW, _L), jnp.int32),
            pltpu.VMEM((2, _R, _L, _D), jnp.float32),
            pltpu.VMEM((2, _R, _L, _D), jnp.float32),
            pltpu.SemaphoreType.DMA,
            pltpu.SemaphoreType.DMA,
            pltpu.SemaphoreType.DMA,
            pltpu.SemaphoreType.DMA,
        ],
    )
    return f(ids, emb_table, glove_table)


def kernel(input_ids, emb_table, glove_table):
    return _run(input_ids.astype(jnp.int32), emb_table, glove_table)
